# baseline (speedup 1.0000x reference)
import functools

import jax
import jax.numpy as jnp
from jax import lax
from jax.experimental import pallas as pl
from jax.experimental.pallas import tpu as pltpu
from jax.experimental.pallas import tpu_sc as plsc

N_NODES = 100000
D_FEAT = 128
HIDDEN = 128
EMBED = 64
BATCH = 10000

NW = 32
N_PAD = 100352
B_PAD = 10240
IDX3 = 3 * B_PAD
_BW = B_PAD // NW
_G16 = _BW // 16
_NB_PAD = (16 * 32 + 15 * 24 + 32) * 112

_SC_MESH = dict(core_axis_name="c", subcore_axis_name="s")


def _worker_id():
    return lax.axis_index("s") * 2 + lax.axis_index("c")


def _make_gather_sum(total_rows, n_chunk, d, ring, dtype, k0_chunks, k1_chunks):
    assert 16 * (k0_chunks + k1_chunks) * n_chunk == total_rows
    assert k0_chunks % ring == 0 and k1_chunks % ring == 0
    max_rows = k0_chunks * n_chunk

    @functools.partial(
        pl.kernel,
        out_type=jax.ShapeDtypeStruct((total_rows, d), dtype),
        mesh=plsc.VectorSubcoreMesh(**_SC_MESH),
        scratch_types=[
            pltpu.VMEM((max_rows,), jnp.int32),
            pltpu.VMEM((max_rows,), jnp.int32),
            pltpu.VMEM((ring, n_chunk, d), dtype),
        ]
        + [pltpu.SemaphoreType.DMA] * (2 * ring),
    )
    def gather_sum(table_hbm, i0_hbm, i1_hbm, out_hbm, i0_v, i1_v, buf_v, *sems):
        gsem, ssem = sems[:ring], sems[ring:]
        c = lax.axis_index("c")
        s = lax.axis_index("s")
        base_chunk = jnp.where(c == 0, s * k0_chunks, 16 * k0_chunks + s * k1_chunks)
        n_outer = jnp.where(c == 0, k0_chunks // ring, k1_chunks // ring)
        wbase = base_chunk * n_chunk
        pltpu.sync_copy(i0_hbm.at[pl.ds(wbase, max_rows)], i0_v)
        pltpu.sync_copy(i1_hbm.at[pl.ds(wbase, max_rows)], i1_v)

        def outer(kk, carry):
            k0 = kk * ring
            g0 = [
                pltpu.async_copy(
                    table_hbm.at[i0_v.at[pl.ds((k0 + s) * n_chunk, n_chunk)]],
                    buf_v.at[s],
                    gsem[s],
                )
                for s in range(ring)
            ]
            g1 = []
            for s in range(ring):
                g0[s].wait()
                g1.append(
                    pltpu.async_copy(
                        table_hbm.at[i1_v.at[pl.ds((k0 + s) * n_chunk, n_chunk)]],
                        buf_v.at[s],
                        gsem[s],
                        add=True,
                    )
                )
            st = []
            for s in range(ring):
                g1[s].wait()
                st.append(
                    pltpu.async_copy(
                        buf_v.at[s],
                        out_hbm.at[pl.ds(wbase + (k0 + s) * n_chunk, n_chunk)],
                        ssem[s],
                    )
                )
            for s in range(ring):
                st[s].wait()
            return carry

        lax.fori_loop(0, n_outer, outer, 0)

    return gather_sum


def _make_gather_sum_combine(total_rows, n_chunk, d, ring, k0_chunks, k1_chunks):
    assert 16 * (k0_chunks + k1_chunks) * n_chunk == total_rows
    assert k0_chunks % ring == 0 and k1_chunks % ring == 0
    max_rows = k0_chunks * n_chunk

    @functools.partial(
        pl.kernel,
        out_type=jax.ShapeDtypeStruct((total_rows, d), jnp.float32),
        mesh=plsc.VectorSubcoreMesh(**_SC_MESH),
        scratch_types=[
            pltpu.VMEM((max_rows,), jnp.int32),
            pltpu.VMEM((max_rows,), jnp.int32),
            pltpu.VMEM((ring, n_chunk, d), jnp.float32),
            pltpu.VMEM((ring, n_chunk, d), jnp.float32),
        ]
        + [pltpu.SemaphoreType.DMA] * (3 * ring),
    )
    def gather_sum_combine(table_hbm, s_hbm, i0_hbm, i1_hbm, out_hbm,
                           i0_v, i1_v, buf_v, sbuf_v, *sems):
        gsem = sems[:ring]
        lsem = sems[ring : 2 * ring]
        ssem = sems[2 * ring :]
        c = lax.axis_index("c")
        sid = lax.axis_index("s")
        base_chunk = jnp.where(c == 0, sid * k0_chunks, 16 * k0_chunks + sid * k1_chunks)
        n_outer = jnp.where(c == 0, k0_chunks // ring, k1_chunks // ring)
        wbase = base_chunk * n_chunk
        pltpu.sync_copy(i0_hbm.at[pl.ds(wbase, max_rows)], i0_v)
        pltpu.sync_copy(i1_hbm.at[pl.ds(wbase, max_rows)], i1_v)

        def outer(kk, carry):
            k0 = kk * ring
            g0, sl_ = [], []
            for s in range(ring):
                row0 = (k0 + s) * n_chunk
                g0.append(
                    pltpu.async_copy(
                        table_hbm.at[i0_v.at[pl.ds(row0, n_chunk)]],
                        buf_v.at[s],
                        gsem[s],
                    )
                )
                sl_.append(
                    pltpu.async_copy(
                        s_hbm.at[pl.ds(wbase + row0, n_chunk)],
                        sbuf_v.at[s],
                        lsem[s],
                    )
                )
            g1 = []
            for s in range(ring):
                g0[s].wait()
                g1.append(
                    pltpu.async_copy(
                        table_hbm.at[i1_v.at[pl.ds((k0 + s) * n_chunk, n_chunk)]],
                        buf_v.at[s],
                        gsem[s],
                        add=True,
                    )
                )
            st = []
            for s in range(ring):
                g1[s].wait()
                sl_[s].wait()

                def combine_row(r, c, s=s):
                    for j in range(d // 16):
                        cols = pl.ds(j * 16, 16)
                        m16 = buf_v[s, r, cols]
                        s16 = sbuf_v[s, r, cols]
                        buf_v[s, r, cols] = jnp.maximum(s16 + 0.5 * m16, 0.0)
                    return c

                lax.fori_loop(0, n_chunk, combine_row, 0)
                st.append(
                    pltpu.async_copy(
                        buf_v.at[s],
                        out_hbm.at[pl.ds(wbase + (k0 + s) * n_chunk, n_chunk)],
                        ssem[s],
                    )
                )
            for s in range(ring):
                st[s].wait()
            return carry

        lax.fori_loop(0, n_outer, outer, 0)

    return gather_sum_combine


def _make_frontier_gather(total_rows, n_chunk, d, ring, dtype):
    rows_per_worker = total_rows // NW
    n_steps = rows_per_worker // n_chunk
    assert n_steps % ring == 0

    @functools.partial(
        pl.kernel,
        out_type=[
            jax.ShapeDtypeStruct((total_rows, d), dtype),
            jax.ShapeDtypeStruct((total_rows, d), dtype),
        ],
        mesh=plsc.VectorSubcoreMesh(**_SC_MESH),
        scratch_types=[
            pltpu.VMEM((rows_per_worker,), jnp.int32),
            pltpu.VMEM((rows_per_worker,), jnp.int32),
            pltpu.VMEM((rows_per_worker,), jnp.int32),
            pltpu.VMEM((ring, n_chunk, d), dtype),
            pltpu.VMEM((ring, n_chunk, d), dtype),
        ]
        + [pltpu.SemaphoreType.DMA] * (4 * ring),
    )
    def frontier_gather(h_hbm, i_hbm, b0_hbm, b1_hbm, gs_hbm, gsum_hbm,
                        i_v, b0_v, b1_v, s_buf, m_buf, *sems):
        sem_s = sems[:ring]
        sem_m = sems[ring : 2 * ring]
        sem_ws = sems[2 * ring : 3 * ring]
        sem_wm = sems[3 * ring :]
        wid = _worker_id()
        wbase = wid * rows_per_worker
        pltpu.sync_copy(i_hbm.at[pl.ds(wbase, rows_per_worker)], i_v)
        pltpu.sync_copy(b0_hbm.at[pl.ds(wbase, rows_per_worker)], b0_v)
        pltpu.sync_copy(b1_hbm.at[pl.ds(wbase, rows_per_worker)], b1_v)

        def outer(kk, carry):
            k0 = kk * ring
            cs, c0 = [], []
            for s in range(ring):
                sl = pl.ds((k0 + s) * n_chunk, n_chunk)
                cs.append(pltpu.async_copy(h_hbm.at[i_v.at[sl]], s_buf.at[s], sem_s[s]))
                c0.append(pltpu.async_copy(h_hbm.at[b0_v.at[sl]], m_buf.at[s], sem_m[s]))
            c1 = []
            for s in range(ring):
                c0[s].wait()
                sl = pl.ds((k0 + s) * n_chunk, n_chunk)
                c1.append(
                    pltpu.async_copy(h_hbm.at[b1_v.at[sl]], m_buf.at[s], sem_m[s], add=True)
                )
            ws, wm = [], []
            for s in range(ring):
                out_sl = pl.ds(wbase + (k0 + s) * n_chunk, n_chunk)
                cs[s].wait()
                ws.append(pltpu.async_copy(s_buf.at[s], gs_hbm.at[out_sl], sem_ws[s]))
                c1[s].wait()
                wm.append(pltpu.async_copy(m_buf.at[s], gsum_hbm.at[out_sl], sem_wm[s]))
            for s in range(ring):
                ws[s].wait()
                wm[s].wait()
            return carry

        lax.fori_loop(0, n_steps // ring, outer, 0)

    return frontier_gather


@functools.partial(
    pl.kernel,
    out_type=[
        jax.ShapeDtypeStruct((IDX3,), jnp.int32),
        jax.ShapeDtypeStruct((IDX3,), jnp.int32),
        jax.ShapeDtypeStruct((IDX3,), jnp.int32),
    ],
    mesh=plsc.VectorSubcoreMesh(**_SC_MESH),
    scratch_types=[
        pltpu.VMEM((N_PAD,), jnp.int32),
        pltpu.VMEM((_BW,), jnp.int32),
        pltpu.VMEM((_BW,), jnp.int32),
        pltpu.VMEM((_BW,), jnp.int32),
        pltpu.VMEM((_BW,), jnp.int32),
        pltpu.VMEM((_BW,), jnp.int32),
        pltpu.VMEM((_BW,), jnp.int32),
        pltpu.VMEM((_BW,), jnp.int32),
    ],
    compiler_params=pltpu.CompilerParams(needs_layout_passes=False),
)
def _index_prep(n_hbm, nb0_hbm, nb1_hbm, idx3_hbm, b0_hbm, b1_hbm,
                table_v, n_v, a0_v, a1_v, c00_v, c01_v, c10_v, c11_v):
    wid = _worker_id()
    base = wid * _BW
    pltpu.sync_copy(n_hbm.at[pl.ds(base, _BW)], n_v)

    def gath(dst_v, src_v):
        def step(g, carry):
            idx = src_v[pl.ds(g * 16, 16)]
            dst_v[pl.ds(g * 16, 16)] = plsc.load_gather(table_v, [idx])
            return carry
        lax.fori_loop(0, _G16, step, 0)

    pltpu.sync_copy(nb0_hbm.at[pl.ds(0, N_PAD)], table_v)
    gath(a0_v, n_v)
    gath(c00_v, a0_v)
    pltpu.sync_copy(nb1_hbm.at[pl.ds(0, N_PAD)], table_v)
    gath(a1_v, n_v)
    gath(c10_v, a0_v)
    gath(c11_v, a1_v)
    pltpu.sync_copy(nb0_hbm.at[pl.ds(0, N_PAD)], table_v)
    gath(c01_v, a1_v)

    for seg, (xi, xb0, xb1) in enumerate(
        [(n_v, a0_v, a1_v), (a0_v, c00_v, c10_v), (a1_v, c01_v, c11_v)]
    ):
        off = seg * B_PAD + base
        pltpu.sync_copy(xi, idx3_hbm.at[pl.ds(off, _BW)])
        pltpu.sync_copy(xb0, b0_hbm.at[pl.ds(off, _BW)])
        pltpu.sync_copy(xb1, b1_hbm.at[pl.ds(off, _BW)])


_BM = 1024


def _mm_first(x, wcat, d_in, d_out):
    def body(x_ref, wcat_ref, s_ref, p_ref):
        xb = x_ref[...].astype(jnp.bfloat16)
        r = jnp.dot(xb, wcat_ref[...], preferred_element_type=jnp.float32)
        s_ref[...] = r[:, :d_out].astype(jnp.bfloat16)
        p_ref[...] = r[:, d_out:]

    return pl.pallas_call(
        body,
        grid=(N_PAD // _BM,),
        in_specs=[
            pl.BlockSpec((_BM, d_in), lambda i: (i, 0)),
            pl.BlockSpec((d_in, 2 * d_out), lambda i: (0, 0)),
        ],
        out_specs=[
            pl.BlockSpec((_BM, d_out), lambda i: (i, 0)),
            pl.BlockSpec((_BM, d_out), lambda i: (i, 0)),
        ],
        out_shape=[
            jax.ShapeDtypeStruct((N_PAD, d_out), jnp.bfloat16),
            jax.ShapeDtypeStruct((N_PAD, d_out), jnp.float32),
        ],
    )(x, wcat)


def _mm_mid(s_in, m_in, wcat, d_in, d_out):
    def body(s_in_ref, m_in_ref, wcat_ref, s_ref, p_ref):
        s = s_in_ref[...].astype(jnp.float32)
        h = jnp.maximum(s + 0.5 * m_in_ref[...], 0.0).astype(jnp.bfloat16)
        r = jnp.dot(h, wcat_ref[...], preferred_element_type=jnp.float32)
        s_ref[...] = r[:, :d_out]
        p_ref[...] = r[:, d_out:]

    return pl.pallas_call(
        body,
        grid=(N_PAD // _BM,),
        in_specs=[
            pl.BlockSpec((_BM, d_in), lambda i: (i, 0)),
            pl.BlockSpec((_BM, d_in), lambda i: (i, 0)),
            pl.BlockSpec((d_in, 2 * d_out), lambda i: (0, 0)),
        ],
        out_specs=[
            pl.BlockSpec((_BM, d_out), lambda i: (i, 0)),
            pl.BlockSpec((_BM, d_out), lambda i: (i, 0)),
        ],
        out_shape=[
            jax.ShapeDtypeStruct((N_PAD, d_out), jnp.float32),
            jax.ShapeDtypeStruct((N_PAD, d_out), jnp.float32),
        ],
    )(s_in, m_in, wcat)


def _combine(s, m, d):
    def body(s_ref, m_ref, h_ref):
        s32 = s_ref[...].astype(jnp.float32)
        h_ref[...] = jnp.maximum(s32 + 0.5 * m_ref[...], 0.0)

    return pl.pallas_call(
        body,
        grid=(N_PAD // _BM,),
        in_specs=[
            pl.BlockSpec((_BM, d), lambda i: (i, 0)),
            pl.BlockSpec((_BM, d), lambda i: (i, 0)),
        ],
        out_specs=pl.BlockSpec((_BM, d), lambda i: (i, 0)),
        out_shape=jax.ShapeDtypeStruct((N_PAD, d), jnp.float32),
    )(s, m)


def _l3(gs, gsum, wcat):
    def body(gs_ref, gsum_ref, wcat_ref, h_ref):
        c = jnp.concatenate(
            [gs_ref[...], 0.5 * gsum_ref[...]], axis=1
        ).astype(jnp.bfloat16)
        r = jnp.dot(c, wcat_ref[...], preferred_element_type=jnp.float32)
        h_ref[...] = jnp.maximum(r, 0.0).astype(jnp.bfloat16)

    return pl.pallas_call(
        body,
        grid=(IDX3 // _BM,),
        in_specs=[
            pl.BlockSpec((_BM, HIDDEN), lambda i: (i, 0)),
            pl.BlockSpec((_BM, HIDDEN), lambda i: (i, 0)),
            pl.BlockSpec((2 * HIDDEN, HIDDEN), lambda i: (0, 0)),
        ],
        out_specs=pl.BlockSpec((_BM, HIDDEN), lambda i: (i, 0)),
        out_shape=jax.ShapeDtypeStruct((IDX3, HIDDEN), jnp.bfloat16),
    )(gs, gsum, wcat)


def _l4(h3p, wcat):
    def body(hs_ref, h0_ref, h1_ref, wcat_ref, out_ref):
        nm = 0.5 * (h0_ref[...].astype(jnp.float32) + h1_ref[...].astype(jnp.float32))
        c = jnp.concatenate(
            [hs_ref[...], nm.astype(jnp.bfloat16)], axis=1
        )
        r = jnp.dot(c, wcat_ref[...], preferred_element_type=jnp.float32)
        out_ref[...] = jnp.maximum(r, 0.0)

    nblk = B_PAD // _BM
    return pl.pallas_call(
        body,
        grid=(nblk,),
        in_specs=[
            pl.BlockSpec((_BM, HIDDEN), lambda i: (i, 0)),
            pl.BlockSpec((_BM, HIDDEN), lambda i: (i + nblk, 0)),
            pl.BlockSpec((_BM, HIDDEN), lambda i: (i + 2 * nblk, 0)),
            pl.BlockSpec((2 * HIDDEN, EMBED), lambda i: (0, 0)),
        ],
        out_specs=pl.BlockSpec((_BM, EMBED), lambda i: (i, 0)),
        out_shape=jax.ShapeDtypeStruct((B_PAD, EMBED), jnp.float32),
    )(h3p, h3p, h3p, wcat)


_gather_sum_full = _make_gather_sum(
    N_PAD, 112, HIDDEN, ring=4, dtype=jnp.float32, k0_chunks=32, k1_chunks=24
)
_gather_sum_combine_full = _make_gather_sum_combine(
    N_PAD, 112, HIDDEN, ring=4, k0_chunks=32, k1_chunks=24
)
_frontier_gather = _make_frontier_gather(IDX3, 96, HIDDEN, ring=2, dtype=jnp.float32)


@jax.jit
def _run(n, x, neighbors, W1, W2, W3, W4):
    nb0 = jnp.pad(neighbors[:, 0], (0, _NB_PAD - N_NODES))
    nb1 = jnp.pad(neighbors[:, 1], (0, _NB_PAD - N_NODES))
    n_p = jnp.pad(n, (0, B_PAD - BATCH))

    wcat1 = W1.T.reshape(2, D_FEAT, HIDDEN).transpose(1, 0, 2).reshape(D_FEAT, 2 * HIDDEN).astype(jnp.bfloat16)
    wcat2 = W2.T.reshape(2, HIDDEN, HIDDEN).transpose(1, 0, 2).reshape(HIDDEN, 2 * HIDDEN).astype(jnp.bfloat16)
    wcat3 = W3.T.astype(jnp.bfloat16)
    wcat4 = W4.T.astype(jnp.bfloat16)

    idx3, b0, b1 = _index_prep(n_p, nb0, nb1)

    s1, p1 = _mm_first(x, wcat1, D_FEAT, HIDDEN)
    m1 = _gather_sum_full(p1, nb0, nb1)
    s2, p2 = _mm_mid(s1, m1, wcat2, HIDDEN, HIDDEN)
    h2 = _gather_sum_combine_full(p2, s2, nb0, nb1)

    gs, gsum = _frontier_gather(h2, idx3, b0, b1)
    h3p = _l3(gs, gsum, wcat3)
    out = _l4(h3p, wcat4)
    return out[:BATCH]


def kernel(n, x, neighbors, W1, W2, W3, W4):
    return _run(n, x, neighbors, W1, W2, W3, W4)


# speedup vs baseline: 1.0024x; 1.0024x over previous
import functools

import jax
import jax.numpy as jnp
from jax import lax
from jax.experimental import pallas as pl
from jax.experimental.pallas import tpu as pltpu
from jax.experimental.pallas import tpu_sc as plsc

N_NODES = 100000
D_FEAT = 128
HIDDEN = 128
EMBED = 64
BATCH = 10000

NW = 32
N_PAD = 100352
B_PAD = 10240
IDX3 = 3 * B_PAD
_BW = B_PAD // NW
_G16 = _BW // 16
_NB_PAD = (16 * 32 + 15 * 24 + 32) * 112

_SC_MESH = dict(core_axis_name="c", subcore_axis_name="s")


def _worker_id():
    return lax.axis_index("s") * 2 + lax.axis_index("c")


def _make_gather_sum(total_rows, n_chunk, d, ring, dtype, k0_chunks, k1_chunks):
    assert 16 * (k0_chunks + k1_chunks) * n_chunk == total_rows
    assert k0_chunks % ring == 0 and k1_chunks % ring == 0
    max_rows = k0_chunks * n_chunk

    @functools.partial(
        pl.kernel,
        out_type=jax.ShapeDtypeStruct((total_rows, d), dtype),
        mesh=plsc.VectorSubcoreMesh(**_SC_MESH),
        scratch_types=[
            pltpu.VMEM((max_rows,), jnp.int32),
            pltpu.VMEM((max_rows,), jnp.int32),
            pltpu.VMEM((ring, n_chunk, d), dtype),
        ]
        + [pltpu.SemaphoreType.DMA] * (2 * ring),
    )
    def gather_sum(table_hbm, i0_hbm, i1_hbm, out_hbm, i0_v, i1_v, buf_v, *sems):
        gsem, ssem = sems[:ring], sems[ring:]
        c = lax.axis_index("c")
        s = lax.axis_index("s")
        base_chunk = jnp.where(c == 0, s * k0_chunks, 16 * k0_chunks + s * k1_chunks)
        n_outer = jnp.where(c == 0, k0_chunks // ring, k1_chunks // ring)
        wbase = base_chunk * n_chunk
        pltpu.sync_copy(i0_hbm.at[pl.ds(wbase, max_rows)], i0_v)
        pltpu.sync_copy(i1_hbm.at[pl.ds(wbase, max_rows)], i1_v)

        def outer(kk, carry):
            k0 = kk * ring
            g0 = [
                pltpu.async_copy(
                    table_hbm.at[i0_v.at[pl.ds((k0 + s) * n_chunk, n_chunk)]],
                    buf_v.at[s],
                    gsem[s],
                )
                for s in range(ring)
            ]
            g1 = []
            for s in range(ring):
                g0[s].wait()
                g1.append(
                    pltpu.async_copy(
                        table_hbm.at[i1_v.at[pl.ds((k0 + s) * n_chunk, n_chunk)]],
                        buf_v.at[s],
                        gsem[s],
                        add=True,
                    )
                )
            st = []
            for s in range(ring):
                g1[s].wait()
                st.append(
                    pltpu.async_copy(
                        buf_v.at[s],
                        out_hbm.at[pl.ds(wbase + (k0 + s) * n_chunk, n_chunk)],
                        ssem[s],
                    )
                )
            for s in range(ring):
                st[s].wait()
            return carry

        lax.fori_loop(0, n_outer, outer, 0)

    return gather_sum


def _make_gather_sum_combine(total_rows, n_chunk, d, ring, k0_chunks, k1_chunks):
    assert 16 * (k0_chunks + k1_chunks) * n_chunk == total_rows
    assert k0_chunks % ring == 0 and k1_chunks % ring == 0
    max_rows = k0_chunks * n_chunk

    @functools.partial(
        pl.kernel,
        out_type=jax.ShapeDtypeStruct((total_rows, d), jnp.float32),
        mesh=plsc.VectorSubcoreMesh(**_SC_MESH),
        scratch_types=[
            pltpu.VMEM((max_rows,), jnp.int32),
            pltpu.VMEM((max_rows,), jnp.int32),
            pltpu.VMEM((ring, n_chunk, d), jnp.float32),
            pltpu.VMEM((ring, n_chunk, d), jnp.float32),
        ]
        + [pltpu.SemaphoreType.DMA] * (3 * ring),
    )
    def gather_sum_combine(table_hbm, s_hbm, i0_hbm, i1_hbm, out_hbm,
                           i0_v, i1_v, buf_v, sbuf_v, *sems):
        gsem = sems[:ring]
        lsem = sems[ring : 2 * ring]
        ssem = sems[2 * ring :]
        c = lax.axis_index("c")
        sid = lax.axis_index("s")
        base_chunk = jnp.where(c == 0, sid * k0_chunks, 16 * k0_chunks + sid * k1_chunks)
        n_outer = jnp.where(c == 0, k0_chunks // ring, k1_chunks // ring)
        wbase = base_chunk * n_chunk
        pltpu.sync_copy(i0_hbm.at[pl.ds(wbase, max_rows)], i0_v)
        pltpu.sync_copy(i1_hbm.at[pl.ds(wbase, max_rows)], i1_v)

        def outer(kk, carry):
            k0 = kk * ring
            g0, sl_ = [], []
            for s in range(ring):
                row0 = (k0 + s) * n_chunk
                g0.append(
                    pltpu.async_copy(
                        table_hbm.at[i0_v.at[pl.ds(row0, n_chunk)]],
                        buf_v.at[s],
                        gsem[s],
                    )
                )
                sl_.append(
                    pltpu.async_copy(
                        s_hbm.at[pl.ds(wbase + row0, n_chunk)],
                        sbuf_v.at[s],
                        lsem[s],
                    )
                )
            g1 = []
            for s in range(ring):
                g0[s].wait()
                g1.append(
                    pltpu.async_copy(
                        table_hbm.at[i1_v.at[pl.ds((k0 + s) * n_chunk, n_chunk)]],
                        buf_v.at[s],
                        gsem[s],
                        add=True,
                    )
                )
            st = []
            for s in range(ring):
                g1[s].wait()
                sl_[s].wait()

                def combine_row(r, c, s=s):
                    for j in range(d // 16):
                        cols = pl.ds(j * 16, 16)
                        m16 = buf_v[s, r, cols]
                        s16 = sbuf_v[s, r, cols]
                        buf_v[s, r, cols] = jnp.maximum(s16 + 0.5 * m16, 0.0)
                    return c

                lax.fori_loop(0, n_chunk, combine_row, 0)
                st.append(
                    pltpu.async_copy(
                        buf_v.at[s],
                        out_hbm.at[pl.ds(wbase + (k0 + s) * n_chunk, n_chunk)],
                        ssem[s],
                    )
                )
            for s in range(ring):
                st[s].wait()
            return carry

        lax.fori_loop(0, n_outer, outer, 0)

    return gather_sum_combine


def _make_frontier_gather(total_rows, n_chunk, d, ring, dtype):
    rows_per_worker = total_rows // NW
    n_steps = rows_per_worker // n_chunk
    assert n_steps % ring == 0

    @functools.partial(
        pl.kernel,
        out_type=[
            jax.ShapeDtypeStruct((total_rows, d), dtype),
            jax.ShapeDtypeStruct((total_rows, d), dtype),
        ],
        mesh=plsc.VectorSubcoreMesh(**_SC_MESH),
        scratch_types=[
            pltpu.VMEM((rows_per_worker,), jnp.int32),
            pltpu.VMEM((rows_per_worker,), jnp.int32),
            pltpu.VMEM((rows_per_worker,), jnp.int32),
            pltpu.VMEM((ring, n_chunk, d), dtype),
            pltpu.VMEM((ring, n_chunk, d), dtype),
        ]
        + [pltpu.SemaphoreType.DMA] * (4 * ring),
    )
    def frontier_gather(h_hbm, i_hbm, b0_hbm, b1_hbm, gs_hbm, gsum_hbm,
                        i_v, b0_v, b1_v, s_buf, m_buf, *sems):
        sem_s = sems[:ring]
        sem_m = sems[ring : 2 * ring]
        sem_ws = sems[2 * ring : 3 * ring]
        sem_wm = sems[3 * ring :]
        wid = _worker_id()
        wbase = wid * rows_per_worker
        pltpu.sync_copy(i_hbm.at[pl.ds(wbase, rows_per_worker)], i_v)
        pltpu.sync_copy(b0_hbm.at[pl.ds(wbase, rows_per_worker)], b0_v)
        pltpu.sync_copy(b1_hbm.at[pl.ds(wbase, rows_per_worker)], b1_v)

        def outer(kk, carry):
            k0 = kk * ring
            cs, c0 = [], []
            for s in range(ring):
                sl = pl.ds((k0 + s) * n_chunk, n_chunk)
                cs.append(pltpu.async_copy(h_hbm.at[i_v.at[sl]], s_buf.at[s], sem_s[s]))
                c0.append(pltpu.async_copy(h_hbm.at[b0_v.at[sl]], m_buf.at[s], sem_m[s]))
            c1 = []
            for s in range(ring):
                c0[s].wait()
                sl = pl.ds((k0 + s) * n_chunk, n_chunk)
                c1.append(
                    pltpu.async_copy(h_hbm.at[b1_v.at[sl]], m_buf.at[s], sem_m[s], add=True)
                )
            ws, wm = [], []
            for s in range(ring):
                out_sl = pl.ds(wbase + (k0 + s) * n_chunk, n_chunk)
                cs[s].wait()
                ws.append(pltpu.async_copy(s_buf.at[s], gs_hbm.at[out_sl], sem_ws[s]))
                c1[s].wait()
                wm.append(pltpu.async_copy(m_buf.at[s], gsum_hbm.at[out_sl], sem_wm[s]))
            for s in range(ring):
                ws[s].wait()
                wm[s].wait()
            return carry

        lax.fori_loop(0, n_steps // ring, outer, 0)

    return frontier_gather


@functools.partial(
    pl.kernel,
    out_type=[
        jax.ShapeDtypeStruct((IDX3,), jnp.int32),
        jax.ShapeDtypeStruct((IDX3,), jnp.int32),
        jax.ShapeDtypeStruct((IDX3,), jnp.int32),
    ],
    mesh=plsc.VectorSubcoreMesh(**_SC_MESH),
    scratch_types=[
        pltpu.VMEM((N_PAD,), jnp.int32),
        pltpu.VMEM((_BW,), jnp.int32),
        pltpu.VMEM((_BW,), jnp.int32),
        pltpu.VMEM((_BW,), jnp.int32),
        pltpu.VMEM((_BW,), jnp.int32),
        pltpu.VMEM((_BW,), jnp.int32),
        pltpu.VMEM((_BW,), jnp.int32),
        pltpu.VMEM((_BW,), jnp.int32),
    ],
    compiler_params=pltpu.CompilerParams(needs_layout_passes=False),
)
def _index_prep(n_hbm, nb0_hbm, nb1_hbm, idx3_hbm, b0_hbm, b1_hbm,
                table_v, n_v, a0_v, a1_v, c00_v, c01_v, c10_v, c11_v):
    wid = _worker_id()
    base = wid * _BW
    pltpu.sync_copy(n_hbm.at[pl.ds(base, _BW)], n_v)

    def gath(dst_v, src_v):
        def step(g, carry):
            idx = src_v[pl.ds(g * 16, 16)]
            dst_v[pl.ds(g * 16, 16)] = plsc.load_gather(table_v, [idx])
            return carry
        lax.fori_loop(0, _G16, step, 0)

    pltpu.sync_copy(nb0_hbm.at[pl.ds(0, N_PAD)], table_v)
    gath(a0_v, n_v)
    gath(c00_v, a0_v)
    pltpu.sync_copy(nb1_hbm.at[pl.ds(0, N_PAD)], table_v)
    gath(a1_v, n_v)
    gath(c10_v, a0_v)
    gath(c11_v, a1_v)
    pltpu.sync_copy(nb0_hbm.at[pl.ds(0, N_PAD)], table_v)
    gath(c01_v, a1_v)

    for seg, (xi, xb0, xb1) in enumerate(
        [(n_v, a0_v, a1_v), (a0_v, c00_v, c10_v), (a1_v, c01_v, c11_v)]
    ):
        off = seg * B_PAD + base
        pltpu.sync_copy(xi, idx3_hbm.at[pl.ds(off, _BW)])
        pltpu.sync_copy(xb0, b0_hbm.at[pl.ds(off, _BW)])
        pltpu.sync_copy(xb1, b1_hbm.at[pl.ds(off, _BW)])


_BM = 1024


def _mm_first(x, wcat, d_in, d_out):
    def body(x_ref, wcat_ref, s_ref, p_ref):
        xb = x_ref[...].astype(jnp.bfloat16)
        r = jnp.dot(xb, wcat_ref[...], preferred_element_type=jnp.float32)
        s_ref[...] = r[:, :d_out].astype(jnp.bfloat16)
        p_ref[...] = r[:, d_out:]

    return pl.pallas_call(
        body,
        grid=(N_PAD // _BM,),
        in_specs=[
            pl.BlockSpec((_BM, d_in), lambda i: (i, 0)),
            pl.BlockSpec((d_in, 2 * d_out), lambda i: (0, 0)),
        ],
        out_specs=[
            pl.BlockSpec((_BM, d_out), lambda i: (i, 0)),
            pl.BlockSpec((_BM, d_out), lambda i: (i, 0)),
        ],
        out_shape=[
            jax.ShapeDtypeStruct((N_PAD, d_out), jnp.bfloat16),
            jax.ShapeDtypeStruct((N_PAD, d_out), jnp.float32),
        ],
    )(x, wcat)


def _mm_mid(s_in, m_in, wcat, d_in, d_out):
    def body(s_in_ref, m_in_ref, wcat_ref, s_ref, p_ref):
        s = s_in_ref[...].astype(jnp.float32)
        h = jnp.maximum(s + 0.5 * m_in_ref[...], 0.0).astype(jnp.bfloat16)
        r = jnp.dot(h, wcat_ref[...], preferred_element_type=jnp.float32)
        s_ref[...] = r[:, :d_out]
        p_ref[...] = r[:, d_out:]

    return pl.pallas_call(
        body,
        grid=(N_PAD // _BM,),
        in_specs=[
            pl.BlockSpec((_BM, d_in), lambda i: (i, 0)),
            pl.BlockSpec((_BM, d_in), lambda i: (i, 0)),
            pl.BlockSpec((d_in, 2 * d_out), lambda i: (0, 0)),
        ],
        out_specs=[
            pl.BlockSpec((_BM, d_out), lambda i: (i, 0)),
            pl.BlockSpec((_BM, d_out), lambda i: (i, 0)),
        ],
        out_shape=[
            jax.ShapeDtypeStruct((N_PAD, d_out), jnp.float32),
            jax.ShapeDtypeStruct((N_PAD, d_out), jnp.float32),
        ],
    )(s_in, m_in, wcat)


def _combine(s, m, d):
    def body(s_ref, m_ref, h_ref):
        s32 = s_ref[...].astype(jnp.float32)
        h_ref[...] = jnp.maximum(s32 + 0.5 * m_ref[...], 0.0)

    return pl.pallas_call(
        body,
        grid=(N_PAD // _BM,),
        in_specs=[
            pl.BlockSpec((_BM, d), lambda i: (i, 0)),
            pl.BlockSpec((_BM, d), lambda i: (i, 0)),
        ],
        out_specs=pl.BlockSpec((_BM, d), lambda i: (i, 0)),
        out_shape=jax.ShapeDtypeStruct((N_PAD, d), jnp.float32),
    )(s, m)


def _l3(gs, gsum, wcat):
    def body(gs_ref, gsum_ref, wcat_ref, h_ref):
        c = jnp.concatenate(
            [gs_ref[...], 0.5 * gsum_ref[...]], axis=1
        ).astype(jnp.bfloat16)
        r = jnp.dot(c, wcat_ref[...], preferred_element_type=jnp.float32)
        h_ref[...] = jnp.maximum(r, 0.0).astype(jnp.bfloat16)

    return pl.pallas_call(
        body,
        grid=(IDX3 // _BM,),
        in_specs=[
            pl.BlockSpec((_BM, HIDDEN), lambda i: (i, 0)),
            pl.BlockSpec((_BM, HIDDEN), lambda i: (i, 0)),
            pl.BlockSpec((2 * HIDDEN, HIDDEN), lambda i: (0, 0)),
        ],
        out_specs=pl.BlockSpec((_BM, HIDDEN), lambda i: (i, 0)),
        out_shape=jax.ShapeDtypeStruct((IDX3, HIDDEN), jnp.bfloat16),
    )(gs, gsum, wcat)


def _l4(h3p, wcat):
    def body(hs_ref, h0_ref, h1_ref, wcat_ref, out_ref):
        nm = 0.5 * (h0_ref[...].astype(jnp.float32) + h1_ref[...].astype(jnp.float32))
        c = jnp.concatenate(
            [hs_ref[...], nm.astype(jnp.bfloat16)], axis=1
        )
        r = jnp.dot(c, wcat_ref[...], preferred_element_type=jnp.float32)
        out_ref[...] = jnp.maximum(r, 0.0)

    nblk = B_PAD // _BM
    return pl.pallas_call(
        body,
        grid=(nblk,),
        in_specs=[
            pl.BlockSpec((_BM, HIDDEN), lambda i: (i, 0)),
            pl.BlockSpec((_BM, HIDDEN), lambda i: (i + nblk, 0)),
            pl.BlockSpec((_BM, HIDDEN), lambda i: (i + 2 * nblk, 0)),
            pl.BlockSpec((2 * HIDDEN, EMBED), lambda i: (0, 0)),
        ],
        out_specs=pl.BlockSpec((_BM, EMBED), lambda i: (i, 0)),
        out_shape=jax.ShapeDtypeStruct((B_PAD, EMBED), jnp.float32),
    )(h3p, h3p, h3p, wcat)


_gather_sum_full = _make_gather_sum(
    N_PAD, 112, HIDDEN, ring=4, dtype=jnp.float32, k0_chunks=28, k1_chunks=28
)
_gather_sum_combine_full = _make_gather_sum_combine(
    N_PAD, 112, HIDDEN, ring=4, k0_chunks=28, k1_chunks=28
)
_frontier_gather = _make_frontier_gather(IDX3, 96, HIDDEN, ring=2, dtype=jnp.float32)


@jax.jit
def _run(n, x, neighbors, W1, W2, W3, W4):
    nb0 = jnp.pad(neighbors[:, 0], (0, _NB_PAD - N_NODES))
    nb1 = jnp.pad(neighbors[:, 1], (0, _NB_PAD - N_NODES))
    n_p = jnp.pad(n, (0, B_PAD - BATCH))

    wcat1 = W1.T.reshape(2, D_FEAT, HIDDEN).transpose(1, 0, 2).reshape(D_FEAT, 2 * HIDDEN).astype(jnp.bfloat16)
    wcat2 = W2.T.reshape(2, HIDDEN, HIDDEN).transpose(1, 0, 2).reshape(HIDDEN, 2 * HIDDEN).astype(jnp.bfloat16)
    wcat3 = W3.T.astype(jnp.bfloat16)
    wcat4 = W4.T.astype(jnp.bfloat16)

    idx3, b0, b1 = _index_prep(n_p, nb0, nb1)

    s1, p1 = _mm_first(x, wcat1, D_FEAT, HIDDEN)
    m1 = _gather_sum_full(p1, nb0, nb1)
    s2, p2 = _mm_mid(s1, m1, wcat2, HIDDEN, HIDDEN)
    h2 = _gather_sum_combine_full(p2, s2, nb0, nb1)

    gs, gsum = _frontier_gather(h2, idx3, b0, b1)
    h3p = _l3(gs, gsum, wcat3)
    out = _l4(h3p, wcat4)
    return out[:BATCH]


def kernel(n, x, neighbors, W1, W2, W3, W4):
    return _run(n, x, neighbors, W1, W2, W3, W4)


# speedup vs baseline: 1.0130x; 1.0106x over previous
import functools

import jax
import jax.numpy as jnp
from jax import lax
from jax.experimental import pallas as pl
from jax.experimental.pallas import tpu as pltpu
from jax.experimental.pallas import tpu_sc as plsc

N_NODES = 100000
D_FEAT = 128
HIDDEN = 128
EMBED = 64
BATCH = 10000

NW = 32
N_PAD = 100352
B_PAD = 10240
IDX3 = 3 * B_PAD
_BW = B_PAD // NW
_G16 = _BW // 16
_NB_PAD = (16 * 32 + 15 * 24 + 32) * 112

_SC_MESH = dict(core_axis_name="c", subcore_axis_name="s")


def _worker_id():
    return lax.axis_index("s") * 2 + lax.axis_index("c")


def _make_gather_sum(total_rows, n_chunk, d, ring, dtype, k0_chunks, k1_chunks):
    assert 16 * (k0_chunks + k1_chunks) * n_chunk == total_rows
    assert k0_chunks % ring == 0 and k1_chunks % ring == 0
    max_rows = k0_chunks * n_chunk

    @functools.partial(
        pl.kernel,
        out_type=jax.ShapeDtypeStruct((total_rows, d), dtype),
        mesh=plsc.VectorSubcoreMesh(**_SC_MESH),
        scratch_types=[
            pltpu.VMEM((max_rows,), jnp.int32),
            pltpu.VMEM((max_rows,), jnp.int32),
            pltpu.VMEM((ring, n_chunk, d), dtype),
        ]
        + [pltpu.SemaphoreType.DMA] * (2 * ring),
    )
    def gather_sum(table_hbm, i0_hbm, i1_hbm, out_hbm, i0_v, i1_v, buf_v, *sems):
        gsem, ssem = sems[:ring], sems[ring:]
        c = lax.axis_index("c")
        s = lax.axis_index("s")
        base_chunk = jnp.where(c == 0, s * k0_chunks, 16 * k0_chunks + s * k1_chunks)
        n_outer = jnp.where(c == 0, k0_chunks // ring, k1_chunks // ring)
        wbase = base_chunk * n_chunk
        pltpu.sync_copy(i0_hbm.at[pl.ds(wbase, max_rows)], i0_v)
        pltpu.sync_copy(i1_hbm.at[pl.ds(wbase, max_rows)], i1_v)

        def outer(kk, carry):
            k0 = kk * ring
            g0 = [
                pltpu.async_copy(
                    table_hbm.at[i0_v.at[pl.ds((k0 + s) * n_chunk, n_chunk)]],
                    buf_v.at[s],
                    gsem[s],
                )
                for s in range(ring)
            ]
            g1 = []
            for s in range(ring):
                g0[s].wait()
                g1.append(
                    pltpu.async_copy(
                        table_hbm.at[i1_v.at[pl.ds((k0 + s) * n_chunk, n_chunk)]],
                        buf_v.at[s],
                        gsem[s],
                        add=True,
                    )
                )
            st = []
            for s in range(ring):
                g1[s].wait()
                st.append(
                    pltpu.async_copy(
                        buf_v.at[s],
                        out_hbm.at[pl.ds(wbase + (k0 + s) * n_chunk, n_chunk)],
                        ssem[s],
                    )
                )
            for s in range(ring):
                st[s].wait()
            return carry

        lax.fori_loop(0, n_outer, outer, 0)

    return gather_sum


def _make_gather_sum_combine(total_rows, n_chunk, d, ring, k0_chunks, k1_chunks):
    assert 16 * (k0_chunks + k1_chunks) * n_chunk == total_rows
    assert k0_chunks % ring == 0 and k1_chunks % ring == 0
    max_rows = k0_chunks * n_chunk

    @functools.partial(
        pl.kernel,
        out_type=jax.ShapeDtypeStruct((total_rows, d), jnp.float32),
        mesh=plsc.VectorSubcoreMesh(**_SC_MESH),
        scratch_types=[
            pltpu.VMEM((max_rows,), jnp.int32),
            pltpu.VMEM((max_rows,), jnp.int32),
            pltpu.VMEM((ring, n_chunk, d), jnp.float32),
            pltpu.VMEM((ring, n_chunk, d), jnp.float32),
        ]
        + [pltpu.SemaphoreType.DMA] * (3 * ring),
    )
    def gather_sum_combine(table_hbm, s_hbm, i0_hbm, i1_hbm, out_hbm,
                           i0_v, i1_v, buf_v, sbuf_v, *sems):
        gsem = sems[:ring]
        lsem = sems[ring : 2 * ring]
        ssem = sems[2 * ring :]
        c = lax.axis_index("c")
        sid = lax.axis_index("s")
        base_chunk = jnp.where(c == 0, sid * k0_chunks, 16 * k0_chunks + sid * k1_chunks)
        n_outer = jnp.where(c == 0, k0_chunks // ring, k1_chunks // ring)
        wbase = base_chunk * n_chunk
        pltpu.sync_copy(i0_hbm.at[pl.ds(wbase, max_rows)], i0_v)
        pltpu.sync_copy(i1_hbm.at[pl.ds(wbase, max_rows)], i1_v)

        def outer(kk, carry):
            k0 = kk * ring
            g0, sl_ = [], []
            for s in range(ring):
                row0 = (k0 + s) * n_chunk
                g0.append(
                    pltpu.async_copy(
                        table_hbm.at[i0_v.at[pl.ds(row0, n_chunk)]],
                        buf_v.at[s],
                        gsem[s],
                    )
                )
                sl_.append(
                    pltpu.async_copy(
                        s_hbm.at[pl.ds(wbase + row0, n_chunk)],
                        sbuf_v.at[s],
                        lsem[s],
                    )
                )
            g1 = []
            for s in range(ring):
                g0[s].wait()
                g1.append(
                    pltpu.async_copy(
                        table_hbm.at[i1_v.at[pl.ds((k0 + s) * n_chunk, n_chunk)]],
                        buf_v.at[s],
                        gsem[s],
                        add=True,
                    )
                )
            st = []
            for s in range(ring):
                g1[s].wait()
                sl_[s].wait()

                def combine_row(r, c, s=s):
                    for j in range(d // 16):
                        cols = pl.ds(j * 16, 16)
                        m16 = buf_v[s, r, cols]
                        s16 = sbuf_v[s, r, cols]
                        buf_v[s, r, cols] = jnp.maximum(s16 + 0.5 * m16, 0.0)
                    return c

                lax.fori_loop(0, n_chunk, combine_row, 0)
                st.append(
                    pltpu.async_copy(
                        buf_v.at[s],
                        out_hbm.at[pl.ds(wbase + (k0 + s) * n_chunk, n_chunk)],
                        ssem[s],
                    )
                )
            for s in range(ring):
                st[s].wait()
            return carry

        lax.fori_loop(0, n_outer, outer, 0)

    return gather_sum_combine


def _make_frontier_gather(total_rows, n_chunk, d, ring, dtype):
    rows_per_worker = total_rows // NW
    n_steps = rows_per_worker // n_chunk
    assert n_steps % ring == 0

    @functools.partial(
        pl.kernel,
        out_type=[
            jax.ShapeDtypeStruct((total_rows, d), dtype),
            jax.ShapeDtypeStruct((total_rows, d), dtype),
        ],
        mesh=plsc.VectorSubcoreMesh(**_SC_MESH),
        scratch_types=[
            pltpu.VMEM((rows_per_worker,), jnp.int32),
            pltpu.VMEM((rows_per_worker,), jnp.int32),
            pltpu.VMEM((rows_per_worker,), jnp.int32),
            pltpu.VMEM((ring, n_chunk, d), dtype),
            pltpu.VMEM((ring, n_chunk, d), dtype),
        ]
        + [pltpu.SemaphoreType.DMA] * (4 * ring),
    )
    def frontier_gather(h_hbm, i_hbm, b0_hbm, b1_hbm, gs_hbm, gsum_hbm,
                        i_v, b0_v, b1_v, s_buf, m_buf, *sems):
        sem_s = sems[:ring]
        sem_m = sems[ring : 2 * ring]
        sem_ws = sems[2 * ring : 3 * ring]
        sem_wm = sems[3 * ring :]
        wid = _worker_id()
        wbase = wid * rows_per_worker
        pltpu.sync_copy(i_hbm.at[pl.ds(wbase, rows_per_worker)], i_v)
        pltpu.sync_copy(b0_hbm.at[pl.ds(wbase, rows_per_worker)], b0_v)
        pltpu.sync_copy(b1_hbm.at[pl.ds(wbase, rows_per_worker)], b1_v)

        def outer(kk, carry):
            k0 = kk * ring
            cs, c0 = [], []
            for s in range(ring):
                sl = pl.ds((k0 + s) * n_chunk, n_chunk)
                cs.append(pltpu.async_copy(h_hbm.at[i_v.at[sl]], s_buf.at[s], sem_s[s]))
                c0.append(pltpu.async_copy(h_hbm.at[b0_v.at[sl]], m_buf.at[s], sem_m[s]))
            c1 = []
            for s in range(ring):
                c0[s].wait()
                sl = pl.ds((k0 + s) * n_chunk, n_chunk)
                c1.append(
                    pltpu.async_copy(h_hbm.at[b1_v.at[sl]], m_buf.at[s], sem_m[s], add=True)
                )
            ws, wm = [], []
            for s in range(ring):
                out_sl = pl.ds(wbase + (k0 + s) * n_chunk, n_chunk)
                cs[s].wait()
                ws.append(pltpu.async_copy(s_buf.at[s], gs_hbm.at[out_sl], sem_ws[s]))
                c1[s].wait()
                wm.append(pltpu.async_copy(m_buf.at[s], gsum_hbm.at[out_sl], sem_wm[s]))
            for s in range(ring):
                ws[s].wait()
                wm[s].wait()
            return carry

        lax.fori_loop(0, n_steps // ring, outer, 0)

    return frontier_gather


@functools.partial(
    pl.kernel,
    out_type=[
        jax.ShapeDtypeStruct((IDX3,), jnp.int32),
        jax.ShapeDtypeStruct((IDX3,), jnp.int32),
        jax.ShapeDtypeStruct((IDX3,), jnp.int32),
    ],
    mesh=plsc.VectorSubcoreMesh(**_SC_MESH),
    scratch_types=[
        pltpu.VMEM((N_PAD,), jnp.int32),
        pltpu.VMEM((_BW,), jnp.int32),
        pltpu.VMEM((_BW,), jnp.int32),
        pltpu.VMEM((_BW,), jnp.int32),
        pltpu.VMEM((_BW,), jnp.int32),
        pltpu.VMEM((_BW,), jnp.int32),
        pltpu.VMEM((_BW,), jnp.int32),
        pltpu.VMEM((_BW,), jnp.int32),
    ],
    compiler_params=pltpu.CompilerParams(needs_layout_passes=False),
)
def _index_prep(n_hbm, nb0_hbm, nb1_hbm, idx3_hbm, b0_hbm, b1_hbm,
                table_v, n_v, a0_v, a1_v, c00_v, c01_v, c10_v, c11_v):
    wid = _worker_id()
    base = wid * _BW
    pltpu.sync_copy(n_hbm.at[pl.ds(base, _BW)], n_v)

    def gath(dst_v, src_v):
        def step(g, carry):
            idx = src_v[pl.ds(g * 16, 16)]
            dst_v[pl.ds(g * 16, 16)] = plsc.load_gather(table_v, [idx])
            return carry
        lax.fori_loop(0, _G16, step, 0)

    pltpu.sync_copy(nb0_hbm.at[pl.ds(0, N_PAD)], table_v)
    gath(a0_v, n_v)
    gath(c00_v, a0_v)
    pltpu.sync_copy(nb1_hbm.at[pl.ds(0, N_PAD)], table_v)
    gath(a1_v, n_v)
    gath(c10_v, a0_v)
    gath(c11_v, a1_v)
    pltpu.sync_copy(nb0_hbm.at[pl.ds(0, N_PAD)], table_v)
    gath(c01_v, a1_v)

    for seg, (xi, xb0, xb1) in enumerate(
        [(n_v, a0_v, a1_v), (a0_v, c00_v, c10_v), (a1_v, c01_v, c11_v)]
    ):
        off = seg * B_PAD + base
        pltpu.sync_copy(xi, idx3_hbm.at[pl.ds(off, _BW)])
        pltpu.sync_copy(xb0, b0_hbm.at[pl.ds(off, _BW)])
        pltpu.sync_copy(xb1, b1_hbm.at[pl.ds(off, _BW)])


_BM = 1024


def _mm_first(x, wcat, d_in, d_out):
    def body(x_ref, wcat_ref, s_ref, p_ref):
        xb = x_ref[...].astype(jnp.bfloat16)
        r = jnp.dot(xb, wcat_ref[...], preferred_element_type=jnp.float32)
        s_ref[...] = r[:, :d_out].astype(jnp.bfloat16)
        p_ref[...] = r[:, d_out:]

    return pl.pallas_call(
        body,
        grid=(N_PAD // _BM,),
        in_specs=[
            pl.BlockSpec((_BM, d_in), lambda i: (i, 0)),
            pl.BlockSpec((d_in, 2 * d_out), lambda i: (0, 0)),
        ],
        out_specs=[
            pl.BlockSpec((_BM, d_out), lambda i: (i, 0)),
            pl.BlockSpec((_BM, d_out), lambda i: (i, 0)),
        ],
        out_shape=[
            jax.ShapeDtypeStruct((N_PAD, d_out), jnp.bfloat16),
            jax.ShapeDtypeStruct((N_PAD, d_out), jnp.float32),
        ],
    )(x, wcat)


def _mm_mid(s_in, m_in, wcat, d_in, d_out):
    def body(s_in_ref, m_in_ref, wcat_ref, s_ref, p_ref):
        s = s_in_ref[...].astype(jnp.float32)
        h = jnp.maximum(s + 0.5 * m_in_ref[...], 0.0).astype(jnp.bfloat16)
        r = jnp.dot(h, wcat_ref[...], preferred_element_type=jnp.float32)
        s_ref[...] = r[:, :d_out]
        p_ref[...] = r[:, d_out:]

    return pl.pallas_call(
        body,
        grid=(N_PAD // _BM,),
        in_specs=[
            pl.BlockSpec((_BM, d_in), lambda i: (i, 0)),
            pl.BlockSpec((_BM, d_in), lambda i: (i, 0)),
            pl.BlockSpec((d_in, 2 * d_out), lambda i: (0, 0)),
        ],
        out_specs=[
            pl.BlockSpec((_BM, d_out), lambda i: (i, 0)),
            pl.BlockSpec((_BM, d_out), lambda i: (i, 0)),
        ],
        out_shape=[
            jax.ShapeDtypeStruct((N_PAD, d_out), jnp.float32),
            jax.ShapeDtypeStruct((N_PAD, d_out), jnp.float32),
        ],
    )(s_in, m_in, wcat)


def _combine(s, m, d):
    def body(s_ref, m_ref, h_ref):
        s32 = s_ref[...].astype(jnp.float32)
        h_ref[...] = jnp.maximum(s32 + 0.5 * m_ref[...], 0.0)

    return pl.pallas_call(
        body,
        grid=(N_PAD // _BM,),
        in_specs=[
            pl.BlockSpec((_BM, d), lambda i: (i, 0)),
            pl.BlockSpec((_BM, d), lambda i: (i, 0)),
        ],
        out_specs=pl.BlockSpec((_BM, d), lambda i: (i, 0)),
        out_shape=jax.ShapeDtypeStruct((N_PAD, d), jnp.float32),
    )(s, m)


def _l3(gs, gsum, wcat):
    def body(gs_ref, gsum_ref, wcat_ref, h_ref):
        c = jnp.concatenate(
            [gs_ref[...], 0.5 * gsum_ref[...]], axis=1
        ).astype(jnp.bfloat16)
        r = jnp.dot(c, wcat_ref[...], preferred_element_type=jnp.float32)
        h_ref[...] = jnp.maximum(r, 0.0).astype(jnp.bfloat16)

    return pl.pallas_call(
        body,
        grid=(IDX3 // _BM,),
        in_specs=[
            pl.BlockSpec((_BM, HIDDEN), lambda i: (i, 0)),
            pl.BlockSpec((_BM, HIDDEN), lambda i: (i, 0)),
            pl.BlockSpec((2 * HIDDEN, HIDDEN), lambda i: (0, 0)),
        ],
        out_specs=pl.BlockSpec((_BM, HIDDEN), lambda i: (i, 0)),
        out_shape=jax.ShapeDtypeStruct((IDX3, HIDDEN), jnp.bfloat16),
    )(gs, gsum, wcat)


def _l4(h3p, wcat):
    def body(hs_ref, h0_ref, h1_ref, wcat_ref, out_ref):
        nm = 0.5 * (h0_ref[...].astype(jnp.float32) + h1_ref[...].astype(jnp.float32))
        c = jnp.concatenate(
            [hs_ref[...], nm.astype(jnp.bfloat16)], axis=1
        )
        r = jnp.dot(c, wcat_ref[...], preferred_element_type=jnp.float32)
        out_ref[...] = jnp.maximum(r, 0.0)

    nblk = B_PAD // _BM
    return pl.pallas_call(
        body,
        grid=(nblk,),
        in_specs=[
            pl.BlockSpec((_BM, HIDDEN), lambda i: (i, 0)),
            pl.BlockSpec((_BM, HIDDEN), lambda i: (i + nblk, 0)),
            pl.BlockSpec((_BM, HIDDEN), lambda i: (i + 2 * nblk, 0)),
            pl.BlockSpec((2 * HIDDEN, EMBED), lambda i: (0, 0)),
        ],
        out_specs=pl.BlockSpec((_BM, EMBED), lambda i: (i, 0)),
        out_shape=jax.ShapeDtypeStruct((B_PAD, EMBED), jnp.float32),
    )(h3p, h3p, h3p, wcat)


_gather_sum_full = _make_gather_sum(
    N_PAD, 112, HIDDEN, ring=7, dtype=jnp.float32, k0_chunks=28, k1_chunks=28
)
_gather_sum_combine_full = _make_gather_sum_combine(
    N_PAD, 112, HIDDEN, ring=4, k0_chunks=28, k1_chunks=28
)
_frontier_gather = _make_frontier_gather(IDX3, 96, HIDDEN, ring=2, dtype=jnp.float32)


@jax.jit
def _run(n, x, neighbors, W1, W2, W3, W4):
    nb0 = jnp.pad(neighbors[:, 0], (0, _NB_PAD - N_NODES))
    nb1 = jnp.pad(neighbors[:, 1], (0, _NB_PAD - N_NODES))
    n_p = jnp.pad(n, (0, B_PAD - BATCH))

    wcat1 = W1.T.reshape(2, D_FEAT, HIDDEN).transpose(1, 0, 2).reshape(D_FEAT, 2 * HIDDEN).astype(jnp.bfloat16)
    wcat2 = W2.T.reshape(2, HIDDEN, HIDDEN).transpose(1, 0, 2).reshape(HIDDEN, 2 * HIDDEN).astype(jnp.bfloat16)
    wcat3 = W3.T.astype(jnp.bfloat16)
    wcat4 = W4.T.astype(jnp.bfloat16)

    idx3, b0, b1 = _index_prep(n_p, nb0, nb1)

    s1, p1 = _mm_first(x, wcat1, D_FEAT, HIDDEN)
    m1 = _gather_sum_full(p1, nb0, nb1)
    s2, p2 = _mm_mid(s1, m1, wcat2, HIDDEN, HIDDEN)
    h2 = _gather_sum_combine_full(p2, s2, nb0, nb1)

    gs, gsum = _frontier_gather(h2, idx3, b0, b1)
    h3p = _l3(gs, gsum, wcat3)
    out = _l4(h3p, wcat4)
    return out[:BATCH]


def kernel(n, x, neighbors, W1, W2, W3, W4):
    return _run(n, x, neighbors, W1, W2, W3, W4)
